# baseline (speedup 1.0000x reference)
import functools

import jax
import jax.numpy as jnp
from jax import lax
from jax.experimental import pallas as pl
from jax.experimental.pallas import tpu as pltpu
from jax.experimental.pallas import tpu_sc as plsc

N = 10000
E = 320000
D = 128
G = 64
NB = 1000
GRID = N // NB

_SELU_ALPHA = 1.6732632423543772
_SELU_SCALE = 1.0507009873554805


def _selu(x):
    return _SELU_SCALE * jnp.where(
        x > 0.0, x, _SELU_ALPHA * (jnp.exp(x) - 1.0))


def _pre_body(x_ref, w_ref, b_ref, a_ref, bb_ref, mi_ref):
    y = jnp.dot(x_ref[...], w_ref[...], preferred_element_type=jnp.float32)
    y = y + b_ref[...]
    a_ref[...] = y[:, :D]
    bb_ref[...] = y[:, D:2 * D]
    mi_ref[...] = y[:, 2 * D:]


def _tc_pre(features, w_cat, b_cat):
    return pl.pallas_call(
        _pre_body,
        grid=(GRID,),
        in_specs=[
            pl.BlockSpec((NB, D), lambda i: (i, 0)),
            pl.BlockSpec((D, 5 * D), lambda i: (0, 0)),
            pl.BlockSpec((1, 5 * D), lambda i: (0, 0)),
        ],
        out_specs=[
            pl.BlockSpec((NB, D), lambda i: (i, 0)),
            pl.BlockSpec((NB, D), lambda i: (i, 0)),
            pl.BlockSpec((NB, 3 * D), lambda i: (i, 0)),
        ],
        out_shape=[
            jax.ShapeDtypeStruct((N, D), jnp.float32),
            jax.ShapeDtypeStruct((N, D), jnp.float32),
            jax.ShapeDtypeStruct((N, 3 * D), jnp.float32),
        ],
    )(features, w_cat, b_cat)


_NC = 2
_NS = 16
_NW = _NC * _NS
_CH = 64
_CHUNKS = 157
_EPT = _CH * _CHUNKS
_EPAD = _NW * _EPT - E
_NP = N + 16
_STRIPE = 624
_TAIL = _NP - _NS * _STRIPE


def _sc_edge_body(first_hbm, second_hbm, a_hbm, b_hbm, zeros_hbm, out_hbm,
                  ra0, ra1, ra2, rb0, rb1, rb2,
                  i10, i11, i12, i13, i20, i21, i22, i23, acc_sh,
                  sa0, sa1, sa2, sb0, sb1, sb2, ss0, ss1, ss2,
                  sj0, sj1, sj2, sj3, sk0, sk1, sk2, sk3):
    c = lax.axis_index("c")
    s = lax.axis_index("s")
    wid = s * _NC + c
    ras = (ra0, ra1, ra2)
    rbs = (rb0, rb1, rb2)
    idx1 = (i10, i11, i12, i13)
    idx2 = (i20, i21, i22, i23)
    sas = (sa0, sa1, sa2)
    sbs = (sb0, sb1, sb2)
    sss = (ss0, ss1, ss2)
    sis = (sj0, sj1, sj2, sj3)
    sks = (sk0, sk1, sk2, sk3)

    r0 = s * _STRIPE
    pltpu.sync_copy(zeros_hbm.at[pl.ds(r0, _STRIPE)],
                    acc_sh.at[pl.ds(r0, _STRIPE)])

    @pl.when(s == _NS - 1)
    def _():
        pltpu.sync_copy(zeros_hbm.at[pl.ds(_NS * _STRIPE, _TAIL)],
                        acc_sh.at[pl.ds(_NS * _STRIPE, _TAIL)])

    plsc.subcore_barrier()

    def i_start(g, q):
        base = wid * _EPT + g * _CH
        pltpu.async_copy(first_hbm.at[pl.ds(base, _CH)], idx1[q], sis[q])
        pltpu.async_copy(second_hbm.at[pl.ds(base, _CH)], idx2[q], sks[q])

    def i_wait(g, q):
        base = wid * _EPT + g * _CH
        pltpu.make_async_copy(
            first_hbm.at[pl.ds(base, _CH)], idx1[q], sis[q]).wait()
        pltpu.make_async_copy(
            second_hbm.at[pl.ds(base, _CH)], idx2[q], sks[q]).wait()

    def g_start(k, q):
        pltpu.async_copy(a_hbm.at[idx1[q]], ras[k], sas[k])
        pltpu.async_copy(b_hbm.at[idx2[q]], rbs[k], sbs[k])

    def g_wait(k, q):
        pltpu.make_async_copy(a_hbm.at[idx1[q]], ras[k], sas[k]).wait()
        pltpu.make_async_copy(b_hbm.at[idx2[q]], rbs[k], sbs[k]).wait()

    def s_start(k, q):
        pltpu.async_copy(ras[k], acc_sh.at[idx2[q]], sss[k], add=True)

    def s_wait(k, q):
        pltpu.make_async_copy(ras[k], acc_sh.at[idx2[q]], sss[k]).wait()

    def compute(k):
        ra, rb = ras[k], rbs[k]

        def row(i, cc):
            for j in range(D // 16):
                sl = pl.ds(j * 16, 16)
                ra[i, sl] = _selu(ra[i, sl] + rb[i, sl])
            return cc

        lax.fori_loop(0, _CH, row, 0, unroll=4)

    def stage(g, k, q):
        kn = (k + 1) % 3

        @pl.when(g >= 2)
        def _():
            s_wait(kn, (q - 2) % 4)

        @pl.when(g + 1 < _CHUNKS)
        def _():
            i_wait(g + 1, (q + 1) % 4)
            g_start(kn, (q + 1) % 4)

        @pl.when(g + 2 < _CHUNKS)
        def _():
            i_start(g + 2, (q + 2) % 4)

        g_wait(k, q)
        compute(k)
        s_start(k, q)

    i_start(0, 0)
    i_start(1, 1)
    i_wait(0, 0)
    g_start(0, 0)

    def block12(i, cc):
        for k in range(12):
            stage(12 * i + k, k % 3, k % 4)
        return cc

    lax.fori_loop(0, _CHUNKS // 12, block12, 0)
    for g in range(12 * (_CHUNKS // 12), _CHUNKS):
        stage(g, g % 3, g % 4)
    s_wait((_CHUNKS - 2) % 3, (_CHUNKS - 2) % 4)
    s_wait((_CHUNKS - 1) % 3, (_CHUNKS - 1) % 4)
    plsc.subcore_barrier()

    pltpu.sync_copy(acc_sh.at[pl.ds(r0, _STRIPE)],
                    out_hbm.at[c, pl.ds(r0, _STRIPE)])

    @pl.when(s == _NS - 1)
    def _():
        pltpu.sync_copy(acc_sh.at[pl.ds(_NS * _STRIPE, _TAIL)],
                        out_hbm.at[c, pl.ds(_NS * _STRIPE, _TAIL)])


def _sc_edge(first, second, a_tab, b_tab, zeros_nd):
    mesh = plsc.VectorSubcoreMesh(core_axis_name="c", subcore_axis_name="s")
    fn = functools.partial(
        pl.kernel,
        out_type=jax.ShapeDtypeStruct((_NC, _NP, D), jnp.float32),
        mesh=mesh,
        scratch_types=(
            [pltpu.VMEM((_CH, D), jnp.float32)] * 6
            + [pltpu.VMEM((_CH,), jnp.int32)] * 8
            + [pltpu.VMEM_SHARED((_NP, D), jnp.float32)]
            + [pltpu.SemaphoreType.DMA] * 17
        ),
    )(_sc_edge_body)
    return fn(first, second, a_tab, b_tab, zeros_nd)


def _post_body(p_ref, mi_ref, f_ref, gid_ref, gk_ref, ib_ref,
               rw1_ref, rb1_ref, rw2_ref, rb2_ref, rw3_ref, rb3_ref,
               o_ref, acc):
    i = pl.program_id(0)
    x = p_ref[0] + p_ref[1]
    mx = jnp.dot(x, gk_ref[...], preferred_element_type=jnp.float32)
    mx = mx + ib_ref[...]
    mi = mi_ref[...]
    z = jax.nn.sigmoid(mx[:, :D] + mi[:, :D])
    r = jax.nn.sigmoid(mx[:, D:2 * D] + mi[:, D:2 * D])
    hh = jnp.tanh(mx[:, 2 * D:] + r * mi[:, 2 * D:])
    ls = z * f_ref[...] + (1.0 - z) * hh
    gid = gid_ref[0, 0, :]
    mask_t = (lax.broadcasted_iota(jnp.int32, (G, NB), 0)
              == gid[None, :]).astype(jnp.float32)
    contrib = jnp.dot(mask_t, ls, preferred_element_type=jnp.float32)

    @pl.when(i == 0)
    def _():
        acc[...] = contrib

    @pl.when(i > 0)
    def _():
        acc[...] = acc[...] + contrib

    @pl.when(i == pl.num_programs(0) - 1)
    def _():
        pooled = acc[...]
        h1 = _selu(
            jnp.dot(pooled, rw1_ref[...], preferred_element_type=jnp.float32)
            + rb1_ref[...])
        h2 = _selu(
            jnp.dot(h1, rw2_ref[...], preferred_element_type=jnp.float32)
            + rb2_ref[...])
        o_ref[...] = (jnp.dot(h2, rw3_ref[...],
                              preferred_element_type=jnp.float32)
                      + rb3_ref[...])


def _tc_post(parts, mi, features, gids3, gk, ib, rw1, rb1, rw2, rb2,
             rw3p, rb3p):
    R = rw1.shape[1]
    return pl.pallas_call(
        _post_body,
        grid=(GRID,),
        in_specs=[
            pl.BlockSpec((2, NB, D), lambda i: (0, i, 0)),
            pl.BlockSpec((NB, 3 * D), lambda i: (i, 0)),
            pl.BlockSpec((NB, D), lambda i: (i, 0)),
            pl.BlockSpec((1, 1, NB), lambda i: (i, 0, 0)),
            pl.BlockSpec((D, 3 * D), lambda i: (0, 0)),
            pl.BlockSpec((1, 3 * D), lambda i: (0, 0)),
            pl.BlockSpec((D, R), lambda i: (0, 0)),
            pl.BlockSpec((1, R), lambda i: (0, 0)),
            pl.BlockSpec((R, R), lambda i: (0, 0)),
            pl.BlockSpec((1, R), lambda i: (0, 0)),
            pl.BlockSpec((R, D), lambda i: (0, 0)),
            pl.BlockSpec((1, D), lambda i: (0, 0)),
        ],
        out_specs=pl.BlockSpec((G, D), lambda i: (0, 0)),
        out_shape=jax.ShapeDtypeStruct((G, D), jnp.float32),
        scratch_shapes=[pltpu.VMEM((G, D), jnp.float32)],
    )(parts, mi, features, gids3, gk, ib, rw1, rb1, rw2, rb2, rw3p, rb3p)


@jax.jit
def kernel(features, graph_ids, edges_topology, W1, b1, gru_kernel,
           gru_rec_kernel, gru_bias, rW1, rb1, rW2, rb2, rW3, rb3):
    w_cat = jnp.concatenate([W1[:D], W1[D:], gru_rec_kernel], axis=1)
    b_cat = jnp.concatenate(
        [b1, jnp.zeros((D,), jnp.float32), gru_bias[1]]).reshape(1, 5 * D)
    a_tab, b_tab, mi = _tc_pre(features, w_cat, b_cat)

    first = jnp.concatenate(
        [edges_topology[0], jnp.zeros((_EPAD,), jnp.int32)])
    second = jnp.concatenate(
        [edges_topology[1], jnp.full((_EPAD,), N, jnp.int32)])
    zeros_nd = jnp.zeros((_NP, D), jnp.float32)
    parts = _sc_edge(first, second, a_tab, b_tab, zeros_nd)

    gids3 = graph_ids.reshape(GRID, 1, NB)
    ib = gru_bias[0].reshape(1, 3 * D)
    rw3p = jnp.pad(rW3, ((0, 0), (0, D - rW3.shape[1])))
    rb3p = jnp.broadcast_to(rb3.reshape(1, 1), (1, D))
    out = _tc_post(parts, mi, features, gids3, gru_kernel, ib,
                   rW1, rb1.reshape(1, -1), rW2, rb2.reshape(1, -1),
                   rw3p, rb3p)
    return out[:, :1]


# speedup vs baseline: 3.7102x; 3.7102x over previous
import functools

import jax
import jax.numpy as jnp
from jax import lax
from jax.experimental import pallas as pl
from jax.experimental.pallas import tpu as pltpu
from jax.experimental.pallas import tpu_sc as plsc

N = 10000
E = 320000
D = 128
G = 64
NB = 1000
GRID = N // NB

_SELU_ALPHA = 1.6732632423543772
_SELU_SCALE = 1.0507009873554805


def _selu(x):
    return _SELU_SCALE * jnp.where(
        x > 0.0, x, _SELU_ALPHA * (jnp.exp(x) - 1.0))


def _pre_body(x_ref, w_ref, b_ref, a_ref, bb_ref, mi_ref):
    y = jnp.dot(x_ref[...], w_ref[...], preferred_element_type=jnp.float32)
    y = y + b_ref[...]
    a_ref[...] = y[:, :D]
    bb_ref[...] = y[:, D:2 * D]
    mi_ref[...] = y[:, 2 * D:]


def _tc_pre(features, w_cat, b_cat):
    return pl.pallas_call(
        _pre_body,
        grid=(GRID,),
        in_specs=[
            pl.BlockSpec((NB, D), lambda i: (i, 0)),
            pl.BlockSpec((D, 5 * D), lambda i: (0, 0)),
            pl.BlockSpec((1, 5 * D), lambda i: (0, 0)),
        ],
        out_specs=[
            pl.BlockSpec((NB, D), lambda i: (i, 0)),
            pl.BlockSpec((NB, D), lambda i: (i, 0)),
            pl.BlockSpec((NB, 3 * D), lambda i: (i, 0)),
        ],
        out_shape=[
            jax.ShapeDtypeStruct((N, D), jnp.float32),
            jax.ShapeDtypeStruct((N, D), jnp.float32),
            jax.ShapeDtypeStruct((N, 3 * D), jnp.float32),
        ],
    )(features, w_cat, b_cat)


_NC = 2
_NS = 16
_NW = _NC * _NS
_CH = 64
_CHUNKS = 157
_EPT = _CH * _CHUNKS
_EPAD = _NW * _EPT - E
_NP = N + 16
_STRIPE = 624
_TAIL = _NP - _NS * _STRIPE


def _sc_edge_body(idxc_hbm, second_hbm, ab_hbm, zeros_hbm, out_hbm,
                  rr0, rr1, rr2,
                  ic0, ic1, ic2, ic3, i20, i21, i22, i23, acc_sh,
                  sa0, sa1, sa2, ss0, ss1, ss2,
                  sj0, sj1, sj2, sj3, sk0, sk1, sk2, sk3):
    c = lax.axis_index("c")
    s = lax.axis_index("s")
    wid = s * _NC + c
    rrs = (rr0, rr1, rr2)
    idxc = (ic0, ic1, ic2, ic3)
    idx2 = (i20, i21, i22, i23)
    sas = (sa0, sa1, sa2)
    sss = (ss0, ss1, ss2)
    sis = (sj0, sj1, sj2, sj3)
    sks = (sk0, sk1, sk2, sk3)

    r0 = s * _STRIPE
    pltpu.sync_copy(zeros_hbm.at[pl.ds(r0, _STRIPE)],
                    acc_sh.at[pl.ds(r0, _STRIPE)])

    @pl.when(s == _NS - 1)
    def _():
        pltpu.sync_copy(zeros_hbm.at[pl.ds(_NS * _STRIPE, _TAIL)],
                        acc_sh.at[pl.ds(_NS * _STRIPE, _TAIL)])

    plsc.subcore_barrier()

    def i_start(g, q):
        ch = wid * _CHUNKS + g
        pltpu.async_copy(idxc_hbm.at[pl.ds(ch * 2 * _CH, 2 * _CH)],
                         idxc[q], sis[q])
        pltpu.async_copy(second_hbm.at[pl.ds(ch * _CH, _CH)],
                         idx2[q], sks[q])

    def i_wait(g, q):
        ch = wid * _CHUNKS + g
        pltpu.make_async_copy(idxc_hbm.at[pl.ds(ch * 2 * _CH, 2 * _CH)],
                              idxc[q], sis[q]).wait()
        pltpu.make_async_copy(second_hbm.at[pl.ds(ch * _CH, _CH)],
                              idx2[q], sks[q]).wait()

    def g_start(k, q):
        pltpu.async_copy(ab_hbm.at[idxc[q]], rrs[k], sas[k])

    def g_wait(k, q):
        pltpu.make_async_copy(ab_hbm.at[idxc[q]], rrs[k], sas[k]).wait()

    def s_start(k, q):
        pltpu.async_copy(rrs[k].at[pl.ds(0, _CH)], acc_sh.at[idx2[q]],
                         sss[k], add=True)

    def s_wait(k, q):
        pltpu.make_async_copy(rrs[k].at[pl.ds(0, _CH)], acc_sh.at[idx2[q]],
                              sss[k]).wait()

    def compute(k):
        rr = rrs[k]

        def row(i, cc):
            for j in range(D // 16):
                sl = pl.ds(j * 16, 16)
                rr[i, sl] = _selu(rr[i, sl] + rr[i + _CH, sl])
            return cc

        lax.fori_loop(0, _CH, row, 0)

    def stage(g, k, q):
        kn = (k + 1) % 3

        @pl.when(g >= 2)
        def _():
            s_wait(kn, (q - 2) % 4)

        @pl.when(g + 1 < _CHUNKS)
        def _():
            i_wait(g + 1, (q + 1) % 4)
            g_start(kn, (q + 1) % 4)

        @pl.when(g + 2 < _CHUNKS)
        def _():
            i_start(g + 2, (q + 2) % 4)

        g_wait(k, q)
        compute(k)
        s_start(k, q)

    i_start(0, 0)
    i_start(1, 1)
    i_wait(0, 0)
    g_start(0, 0)

    def block12(i, cc):
        for k in range(12):
            stage(12 * i + k, k % 3, k % 4)
        return cc

    lax.fori_loop(0, _CHUNKS // 12, block12, 0)
    for g in range(12 * (_CHUNKS // 12), _CHUNKS):
        stage(g, g % 3, g % 4)
    s_wait((_CHUNKS - 2) % 3, (_CHUNKS - 2) % 4)
    s_wait((_CHUNKS - 1) % 3, (_CHUNKS - 1) % 4)
    plsc.subcore_barrier()

    pltpu.sync_copy(acc_sh.at[pl.ds(r0, _STRIPE)],
                    out_hbm.at[c, pl.ds(r0, _STRIPE)])

    @pl.when(s == _NS - 1)
    def _():
        pltpu.sync_copy(acc_sh.at[pl.ds(_NS * _STRIPE, _TAIL)],
                        out_hbm.at[c, pl.ds(_NS * _STRIPE, _TAIL)])


def _sc_edge(idxc, second, ab_tab, zeros_nd):
    mesh = plsc.VectorSubcoreMesh(core_axis_name="c", subcore_axis_name="s")
    fn = functools.partial(
        pl.kernel,
        out_type=jax.ShapeDtypeStruct((_NC, _NP, D), jnp.float32),
        mesh=mesh,
        scratch_types=(
            [pltpu.VMEM((2 * _CH, D), jnp.float32)] * 3
            + [pltpu.VMEM((2 * _CH,), jnp.int32)] * 4
            + [pltpu.VMEM((_CH,), jnp.int32)] * 4
            + [pltpu.VMEM_SHARED((_NP, D), jnp.float32)]
            + [pltpu.SemaphoreType.DMA] * 14
        ),
    )(_sc_edge_body)
    return fn(idxc, second, ab_tab, zeros_nd)


def _post_body(p_ref, mi_ref, f_ref, gid_ref, gk_ref, ib_ref,
               rw1_ref, rb1_ref, rw2_ref, rb2_ref, rw3_ref, rb3_ref,
               o_ref, acc):
    i = pl.program_id(0)
    x = p_ref[0] + p_ref[1]
    mx = jnp.dot(x, gk_ref[...], preferred_element_type=jnp.float32)
    mx = mx + ib_ref[...]
    mi = mi_ref[...]
    z = jax.nn.sigmoid(mx[:, :D] + mi[:, :D])
    r = jax.nn.sigmoid(mx[:, D:2 * D] + mi[:, D:2 * D])
    hh = jnp.tanh(mx[:, 2 * D:] + r * mi[:, 2 * D:])
    ls = z * f_ref[...] + (1.0 - z) * hh
    gid = gid_ref[0, 0, :]
    mask_t = (lax.broadcasted_iota(jnp.int32, (G, NB), 0)
              == gid[None, :]).astype(jnp.float32)
    contrib = jnp.dot(mask_t, ls, preferred_element_type=jnp.float32)

    @pl.when(i == 0)
    def _():
        acc[...] = contrib

    @pl.when(i > 0)
    def _():
        acc[...] = acc[...] + contrib

    @pl.when(i == pl.num_programs(0) - 1)
    def _():
        pooled = acc[...]
        h1 = _selu(
            jnp.dot(pooled, rw1_ref[...], preferred_element_type=jnp.float32)
            + rb1_ref[...])
        h2 = _selu(
            jnp.dot(h1, rw2_ref[...], preferred_element_type=jnp.float32)
            + rb2_ref[...])
        o_ref[...] = (jnp.dot(h2, rw3_ref[...],
                              preferred_element_type=jnp.float32)
                      + rb3_ref[...])


def _tc_post(parts, mi, features, gids3, gk, ib, rw1, rb1, rw2, rb2,
             rw3p, rb3p):
    R = rw1.shape[1]
    return pl.pallas_call(
        _post_body,
        grid=(GRID,),
        in_specs=[
            pl.BlockSpec((2, NB, D), lambda i: (0, i, 0)),
            pl.BlockSpec((NB, 3 * D), lambda i: (i, 0)),
            pl.BlockSpec((NB, D), lambda i: (i, 0)),
            pl.BlockSpec((1, 1, NB), lambda i: (i, 0, 0)),
            pl.BlockSpec((D, 3 * D), lambda i: (0, 0)),
            pl.BlockSpec((1, 3 * D), lambda i: (0, 0)),
            pl.BlockSpec((D, R), lambda i: (0, 0)),
            pl.BlockSpec((1, R), lambda i: (0, 0)),
            pl.BlockSpec((R, R), lambda i: (0, 0)),
            pl.BlockSpec((1, R), lambda i: (0, 0)),
            pl.BlockSpec((R, D), lambda i: (0, 0)),
            pl.BlockSpec((1, D), lambda i: (0, 0)),
        ],
        out_specs=pl.BlockSpec((G, D), lambda i: (0, 0)),
        out_shape=jax.ShapeDtypeStruct((G, D), jnp.float32),
        scratch_shapes=[pltpu.VMEM((G, D), jnp.float32)],
    )(parts, mi, features, gids3, gk, ib, rw1, rb1, rw2, rb2, rw3p, rb3p)


@jax.jit
def kernel(features, graph_ids, edges_topology, W1, b1, gru_kernel,
           gru_rec_kernel, gru_bias, rW1, rb1, rW2, rb2, rW3, rb3):
    w_cat = jnp.concatenate([W1[:D], W1[D:], gru_rec_kernel], axis=1)
    b_cat = jnp.concatenate(
        [b1, jnp.zeros((D,), jnp.float32), gru_bias[1]]).reshape(1, 5 * D)
    a_tab, b_tab, mi = _tc_pre(features, w_cat, b_cat)

    first = jnp.concatenate(
        [edges_topology[0], jnp.zeros((_EPAD,), jnp.int32)])
    second = jnp.concatenate(
        [edges_topology[1], jnp.full((_EPAD,), N, jnp.int32)])
    idxc = jnp.concatenate(
        [first.reshape(-1, _CH), second.reshape(-1, _CH) + N],
        axis=1).reshape(-1)
    ab_tab = jnp.concatenate(
        [a_tab, b_tab, jnp.zeros((_NP - N, D), jnp.float32)], axis=0)
    zeros_nd = jnp.zeros((_NP, D), jnp.float32)
    parts = _sc_edge(idxc, second, ab_tab, zeros_nd)

    gids3 = graph_ids.reshape(GRID, 1, NB)
    ib = gru_bias[0].reshape(1, 3 * D)
    rw3p = jnp.pad(rW3, ((0, 0), (0, D - rW3.shape[1])))
    rb3p = jnp.broadcast_to(rb3.reshape(1, 1), (1, D))
    out = _tc_post(parts, mi, features, gids3, gru_kernel, ib,
                   rW1, rb1.reshape(1, -1), rW2, rb2.reshape(1, -1),
                   rw3p, rb3p)
    return out[:, :1]


# speedup vs baseline: 3.8424x; 1.0356x over previous
import functools

import jax
import jax.numpy as jnp
from jax import lax
from jax.experimental import pallas as pl
from jax.experimental.pallas import tpu as pltpu
from jax.experimental.pallas import tpu_sc as plsc

N = 10000
E = 320000
D = 128
G = 64
NB = 1000
GRID = N // NB

_SELU_ALPHA = 1.6732632423543772
_SELU_SCALE = 1.0507009873554805


def _selu(x):
    return _SELU_SCALE * jnp.where(
        x > 0.0, x, _SELU_ALPHA * (jnp.exp(x) - 1.0))


def _pre_body(x_ref, w_ref, b_ref, ab_ref, mi_ref):
    y = jnp.dot(x_ref[...], w_ref[...], preferred_element_type=jnp.float32)
    y = y + b_ref[...]
    ab_ref[:, 0, :] = y[:, :D]
    ab_ref[:, 1, :] = y[:, D:2 * D]
    mi_ref[...] = y[:, 2 * D:]


def _tc_pre(features, w_cat, b_cat):
    return pl.pallas_call(
        _pre_body,
        grid=(GRID,),
        in_specs=[
            pl.BlockSpec((NB, D), lambda i: (i, 0)),
            pl.BlockSpec((D, 5 * D), lambda i: (0, 0)),
            pl.BlockSpec((1, 5 * D), lambda i: (0, 0)),
        ],
        out_specs=[
            pl.BlockSpec((NB, 2, D), lambda i: (i, 0, 0)),
            pl.BlockSpec((NB, 3 * D), lambda i: (i, 0)),
        ],
        out_shape=[
            jax.ShapeDtypeStruct((N, 2, D), jnp.float32),
            jax.ShapeDtypeStruct((N, 3 * D), jnp.float32),
        ],
    )(features, w_cat, b_cat)


_NC = 2
_NS = 16
_NW = _NC * _NS
_CH = 64
_CHUNKS = 157
_EPT = _CH * _CHUNKS
_EPAD = _NW * _EPT - E
_NP = N + 16
_STRIPE = 624
_TAIL = _NP - _NS * _STRIPE


def _sc_edge_body(idxc_hbm, second_hbm, ab_hbm, zeros_hbm, out_hbm,
                  rr0, rr1, rr2,
                  ic0, ic1, ic2, ic3, i20, i21, i22, i23, acc_sh,
                  sa0, sa1, sa2, ss0, ss1, ss2,
                  sj0, sj1, sj2, sj3, sk0, sk1, sk2, sk3):
    c = lax.axis_index("c")
    s = lax.axis_index("s")
    wid = s * _NC + c
    rrs = (rr0, rr1, rr2)
    idxc = (ic0, ic1, ic2, ic3)
    idx2 = (i20, i21, i22, i23)
    sas = (sa0, sa1, sa2)
    sss = (ss0, ss1, ss2)
    sis = (sj0, sj1, sj2, sj3)
    sks = (sk0, sk1, sk2, sk3)

    r0 = s * _STRIPE
    pltpu.sync_copy(zeros_hbm.at[pl.ds(r0, _STRIPE)],
                    acc_sh.at[pl.ds(r0, _STRIPE)])

    @pl.when(s == _NS - 1)
    def _():
        pltpu.sync_copy(zeros_hbm.at[pl.ds(_NS * _STRIPE, _TAIL)],
                        acc_sh.at[pl.ds(_NS * _STRIPE, _TAIL)])

    plsc.subcore_barrier()

    def i_start(g, q):
        ch = wid * _CHUNKS + g
        pltpu.async_copy(idxc_hbm.at[pl.ds(ch * 2 * _CH, 2 * _CH)],
                         idxc[q], sis[q])
        pltpu.async_copy(second_hbm.at[pl.ds(ch * _CH, _CH)],
                         idx2[q], sks[q])

    def i_wait(g, q):
        ch = wid * _CHUNKS + g
        pltpu.make_async_copy(idxc_hbm.at[pl.ds(ch * 2 * _CH, 2 * _CH)],
                              idxc[q], sis[q]).wait()
        pltpu.make_async_copy(second_hbm.at[pl.ds(ch * _CH, _CH)],
                              idx2[q], sks[q]).wait()

    def g_start(k, q):
        pltpu.async_copy(ab_hbm.at[idxc[q]], rrs[k], sas[k])

    def g_wait(k, q):
        pltpu.make_async_copy(ab_hbm.at[idxc[q]], rrs[k], sas[k]).wait()

    def s_start(k, q):
        pltpu.async_copy(rrs[k].at[pl.ds(0, _CH)], acc_sh.at[idx2[q]],
                         sss[k], add=True)

    def s_wait(k, q):
        pltpu.make_async_copy(rrs[k].at[pl.ds(0, _CH)], acc_sh.at[idx2[q]],
                              sss[k]).wait()

    def compute(k):
        rr = rrs[k]

        @plsc.parallel_loop(0, _CH)
        def _row(i):
            for j in range(D // 16):
                sl = pl.ds(j * 16, 16)
                rr[i, sl] = _selu(rr[i, sl] + rr[i + _CH, sl])

    def stage(g, k, q):
        kn = (k + 1) % 3

        @pl.when(g >= 2)
        def _():
            s_wait(kn, (q - 2) % 4)

        @pl.when(g + 1 < _CHUNKS)
        def _():
            i_wait(g + 1, (q + 1) % 4)
            g_start(kn, (q + 1) % 4)

        @pl.when(g + 2 < _CHUNKS)
        def _():
            i_start(g + 2, (q + 2) % 4)

        g_wait(k, q)
        compute(k)
        s_start(k, q)

    i_start(0, 0)
    i_start(1, 1)
    i_wait(0, 0)
    g_start(0, 0)

    def block12(i, cc):
        for k in range(12):
            stage(12 * i + k, k % 3, k % 4)
        return cc

    lax.fori_loop(0, _CHUNKS // 12, block12, 0)
    for g in range(12 * (_CHUNKS // 12), _CHUNKS):
        stage(g, g % 3, g % 4)
    s_wait((_CHUNKS - 2) % 3, (_CHUNKS - 2) % 4)
    s_wait((_CHUNKS - 1) % 3, (_CHUNKS - 1) % 4)
    plsc.subcore_barrier()

    pltpu.sync_copy(acc_sh.at[pl.ds(r0, _STRIPE)],
                    out_hbm.at[c, pl.ds(r0, _STRIPE)])

    @pl.when(s == _NS - 1)
    def _():
        pltpu.sync_copy(acc_sh.at[pl.ds(_NS * _STRIPE, _TAIL)],
                        out_hbm.at[c, pl.ds(_NS * _STRIPE, _TAIL)])


def _sc_edge(idxc, second, ab_tab, zeros_nd):
    mesh = plsc.VectorSubcoreMesh(core_axis_name="c", subcore_axis_name="s")
    fn = functools.partial(
        pl.kernel,
        out_type=jax.ShapeDtypeStruct((_NC, _NP, D), jnp.float32),
        mesh=mesh,
        scratch_types=(
            [pltpu.VMEM((2 * _CH, D), jnp.float32)] * 3
            + [pltpu.VMEM((2 * _CH,), jnp.int32)] * 4
            + [pltpu.VMEM((_CH,), jnp.int32)] * 4
            + [pltpu.VMEM_SHARED((_NP, D), jnp.float32)]
            + [pltpu.SemaphoreType.DMA] * 14
        ),
    )(_sc_edge_body)
    return fn(idxc, second, ab_tab, zeros_nd)


def _post_body(p_ref, mi_ref, f_ref, gid_ref, gk_ref, ib_ref,
               rw1_ref, rb1_ref, rw2_ref, rb2_ref, rw3_ref, rb3_ref,
               o_ref, acc):
    i = pl.program_id(0)
    x = p_ref[0] + p_ref[1]
    mx = jnp.dot(x, gk_ref[...], preferred_element_type=jnp.float32)
    mx = mx + ib_ref[...]
    mi = mi_ref[...]
    z = jax.nn.sigmoid(mx[:, :D] + mi[:, :D])
    r = jax.nn.sigmoid(mx[:, D:2 * D] + mi[:, D:2 * D])
    hh = jnp.tanh(mx[:, 2 * D:] + r * mi[:, 2 * D:])
    ls = z * f_ref[...] + (1.0 - z) * hh
    gid = gid_ref[0, 0, :]
    mask_t = (lax.broadcasted_iota(jnp.int32, (G, NB), 0)
              == gid[None, :]).astype(jnp.float32)
    contrib = jnp.dot(mask_t, ls, preferred_element_type=jnp.float32)

    @pl.when(i == 0)
    def _():
        acc[...] = contrib

    @pl.when(i > 0)
    def _():
        acc[...] = acc[...] + contrib

    @pl.when(i == pl.num_programs(0) - 1)
    def _():
        pooled = acc[...]
        h1 = _selu(
            jnp.dot(pooled, rw1_ref[...], preferred_element_type=jnp.float32)
            + rb1_ref[...])
        h2 = _selu(
            jnp.dot(h1, rw2_ref[...], preferred_element_type=jnp.float32)
            + rb2_ref[...])
        o_ref[...] = (jnp.dot(h2, rw3_ref[...],
                              preferred_element_type=jnp.float32)
                      + rb3_ref[...])


def _tc_post(parts, mi, features, gids3, gk, ib, rw1, rb1, rw2, rb2,
             rw3p, rb3p):
    R = rw1.shape[1]
    return pl.pallas_call(
        _post_body,
        grid=(GRID,),
        in_specs=[
            pl.BlockSpec((2, NB, D), lambda i: (0, i, 0)),
            pl.BlockSpec((NB, 3 * D), lambda i: (i, 0)),
            pl.BlockSpec((NB, D), lambda i: (i, 0)),
            pl.BlockSpec((1, 1, NB), lambda i: (i, 0, 0)),
            pl.BlockSpec((D, 3 * D), lambda i: (0, 0)),
            pl.BlockSpec((1, 3 * D), lambda i: (0, 0)),
            pl.BlockSpec((D, R), lambda i: (0, 0)),
            pl.BlockSpec((1, R), lambda i: (0, 0)),
            pl.BlockSpec((R, R), lambda i: (0, 0)),
            pl.BlockSpec((1, R), lambda i: (0, 0)),
            pl.BlockSpec((R, D), lambda i: (0, 0)),
            pl.BlockSpec((1, D), lambda i: (0, 0)),
        ],
        out_specs=pl.BlockSpec((G, D), lambda i: (0, 0)),
        out_shape=jax.ShapeDtypeStruct((G, D), jnp.float32),
        scratch_shapes=[pltpu.VMEM((G, D), jnp.float32)],
    )(parts, mi, features, gids3, gk, ib, rw1, rb1, rw2, rb2, rw3p, rb3p)


@jax.jit
def kernel(features, graph_ids, edges_topology, W1, b1, gru_kernel,
           gru_rec_kernel, gru_bias, rW1, rb1, rW2, rb2, rW3, rb3):
    w_cat = jnp.concatenate([W1[:D], W1[D:], gru_rec_kernel], axis=1)
    b_cat = jnp.concatenate(
        [b1, jnp.zeros((D,), jnp.float32), gru_bias[1]]).reshape(1, 5 * D)
    ab_tab, mi = _tc_pre(features, w_cat, b_cat)

    first = jnp.concatenate(
        [edges_topology[0], jnp.zeros((_EPAD,), jnp.int32)])
    second_g = jnp.concatenate(
        [edges_topology[1], jnp.zeros((_EPAD,), jnp.int32)])
    second = jnp.concatenate(
        [edges_topology[1], jnp.full((_EPAD,), N, jnp.int32)])
    idxc = jnp.concatenate(
        [2 * first.reshape(-1, _CH), 2 * second_g.reshape(-1, _CH) + 1],
        axis=1).reshape(-1)
    zeros_nd = jnp.zeros((_NP, D), jnp.float32)
    parts = _sc_edge(idxc, second, ab_tab.reshape(2 * N, D), zeros_nd)

    gids3 = graph_ids.reshape(GRID, 1, NB)
    ib = gru_bias[0].reshape(1, 3 * D)
    rw3p = jnp.pad(rW3, ((0, 0), (0, D - rW3.shape[1])))
    rb3p = jnp.broadcast_to(rb3.reshape(1, 1), (1, D))
    out = _tc_post(parts, mi, features, gids3, gru_kernel, ib,
                   rW1, rb1.reshape(1, -1), rW2, rb2.reshape(1, -1),
                   rw3p, rb3p)
    return out[:, :1]


# speedup vs baseline: 3.8710x; 1.0075x over previous
import functools

import jax
import jax.numpy as jnp
from jax import lax
from jax.experimental import pallas as pl
from jax.experimental.pallas import tpu as pltpu
from jax.experimental.pallas import tpu_sc as plsc

N = 10000
E = 320000
D = 128
G = 64
NB = 1000
GRID = N // NB

_SELU_ALPHA = 1.6732632423543772
_SELU_SCALE = 1.0507009873554805


def _selu(x):
    return _SELU_SCALE * jnp.where(
        x > 0.0, x, _SELU_ALPHA * (jnp.exp(x) - 1.0))


def _pre_body(x_ref, w_ref, b_ref, ab_ref, mi_ref):
    y = jnp.dot(x_ref[...], w_ref[...], preferred_element_type=jnp.float32)
    y = y + b_ref[...]
    ab_ref[:, 0, :] = y[:, :D]
    ab_ref[:, 1, :] = y[:, D:2 * D]
    mi_ref[...] = y[:, 2 * D:]


def _tc_pre(features, w_cat, b_cat):
    return pl.pallas_call(
        _pre_body,
        grid=(GRID,),
        in_specs=[
            pl.BlockSpec((NB, D), lambda i: (i, 0)),
            pl.BlockSpec((D, 5 * D), lambda i: (0, 0)),
            pl.BlockSpec((1, 5 * D), lambda i: (0, 0)),
        ],
        out_specs=[
            pl.BlockSpec((NB, 2, D), lambda i: (i, 0, 0)),
            pl.BlockSpec((NB, 3 * D), lambda i: (i, 0)),
        ],
        out_shape=[
            jax.ShapeDtypeStruct((N, 2, D), jnp.float32),
            jax.ShapeDtypeStruct((N, 3 * D), jnp.float32),
        ],
    )(features, w_cat, b_cat)


_NC = 2
_NS = 16
_NW = _NC * _NS
_CH = 64
_CHUNKS = 157
_EPT = _CH * _CHUNKS
_EPAD = _NW * _EPT - E
_NP = N + 16
_STRIPE = 624
_TAIL = _NP - _NS * _STRIPE


def _sc_edge_body(idxc_hbm, second_hbm, ab_hbm, zeros_hbm, out_hbm,
                  rr0, rr1, rr2,
                  ic0, ic1, ic2, ic3, i20, i21, i22, i23, acc_sh,
                  sa0, sa1, sa2, ss0, ss1, ss2,
                  sj0, sj1, sj2, sj3, sk0, sk1, sk2, sk3):
    c = lax.axis_index("c")
    s = lax.axis_index("s")
    wid = s * _NC + c
    rrs = (rr0, rr1, rr2)
    idxc = (ic0, ic1, ic2, ic3)
    idx2 = (i20, i21, i22, i23)
    sas = (sa0, sa1, sa2)
    sss = (ss0, ss1, ss2)
    sis = (sj0, sj1, sj2, sj3)
    sks = (sk0, sk1, sk2, sk3)

    r0 = s * _STRIPE
    pltpu.sync_copy(zeros_hbm.at[pl.ds(r0, _STRIPE)],
                    acc_sh.at[pl.ds(r0, _STRIPE)])

    @pl.when(s == _NS - 1)
    def _():
        pltpu.sync_copy(zeros_hbm.at[pl.ds(_NS * _STRIPE, _TAIL)],
                        acc_sh.at[pl.ds(_NS * _STRIPE, _TAIL)])

    plsc.subcore_barrier()

    def i_start(g, q):
        ch = wid * _CHUNKS + g
        pltpu.async_copy(idxc_hbm.at[pl.ds(ch * 2 * _CH, 2 * _CH)],
                         idxc[q], sis[q])
        pltpu.async_copy(second_hbm.at[pl.ds(ch * _CH, _CH)],
                         idx2[q], sks[q])

    def i_wait(g, q):
        ch = wid * _CHUNKS + g
        pltpu.make_async_copy(idxc_hbm.at[pl.ds(ch * 2 * _CH, 2 * _CH)],
                              idxc[q], sis[q]).wait()
        pltpu.make_async_copy(second_hbm.at[pl.ds(ch * _CH, _CH)],
                              idx2[q], sks[q]).wait()

    def g_start(k, q):
        pltpu.async_copy(ab_hbm.at[idxc[q]], rrs[k], sas[k])

    def g_wait(k, q):
        pltpu.make_async_copy(ab_hbm.at[idxc[q]], rrs[k], sas[k]).wait()

    def s_start(k, q):
        pltpu.async_copy(rrs[k].at[pl.ds(0, _CH)], acc_sh.at[idx2[q]],
                         sss[k], add=True)

    def s_wait(k, q):
        pltpu.make_async_copy(rrs[k].at[pl.ds(0, _CH)], acc_sh.at[idx2[q]],
                              sss[k]).wait()

    def compute(k):
        rr = rrs[k]

        @plsc.parallel_loop(0, _CH, 1, unroll=2)
        def _row(i):
            for j in range(D // 16):
                sl = pl.ds(j * 16, 16)
                rr[i, sl] = _selu(rr[i, sl] + rr[i + _CH, sl])

    def stage(g, k, q):
        kn = (k + 1) % 3

        @pl.when(g >= 2)
        def _():
            s_wait(kn, (q - 2) % 4)

        @pl.when(g + 1 < _CHUNKS)
        def _():
            i_wait(g + 1, (q + 1) % 4)
            g_start(kn, (q + 1) % 4)

        @pl.when(g + 2 < _CHUNKS)
        def _():
            i_start(g + 2, (q + 2) % 4)

        g_wait(k, q)
        compute(k)
        s_start(k, q)

    i_start(0, 0)
    i_start(1, 1)
    i_wait(0, 0)
    g_start(0, 0)

    def block12(i, cc):
        for k in range(12):
            stage(12 * i + k, k % 3, k % 4)
        return cc

    lax.fori_loop(0, _CHUNKS // 12, block12, 0)
    for g in range(12 * (_CHUNKS // 12), _CHUNKS):
        stage(g, g % 3, g % 4)
    s_wait((_CHUNKS - 2) % 3, (_CHUNKS - 2) % 4)
    s_wait((_CHUNKS - 1) % 3, (_CHUNKS - 1) % 4)
    plsc.subcore_barrier()

    pltpu.sync_copy(acc_sh.at[pl.ds(r0, _STRIPE)],
                    out_hbm.at[c, pl.ds(r0, _STRIPE)])

    @pl.when(s == _NS - 1)
    def _():
        pltpu.sync_copy(acc_sh.at[pl.ds(_NS * _STRIPE, _TAIL)],
                        out_hbm.at[c, pl.ds(_NS * _STRIPE, _TAIL)])


def _sc_edge(idxc, second, ab_tab, zeros_nd):
    mesh = plsc.VectorSubcoreMesh(core_axis_name="c", subcore_axis_name="s")
    fn = functools.partial(
        pl.kernel,
        out_type=jax.ShapeDtypeStruct((_NC, _NP, D), jnp.float32),
        mesh=mesh,
        scratch_types=(
            [pltpu.VMEM((2 * _CH, D), jnp.float32)] * 3
            + [pltpu.VMEM((2 * _CH,), jnp.int32)] * 4
            + [pltpu.VMEM((_CH,), jnp.int32)] * 4
            + [pltpu.VMEM_SHARED((_NP, D), jnp.float32)]
            + [pltpu.SemaphoreType.DMA] * 14
        ),
    )(_sc_edge_body)
    return fn(idxc, second, ab_tab, zeros_nd)


def _post_body(p_ref, mi_ref, f_ref, gid_ref, gk_ref, ib_ref,
               rw1_ref, rb1_ref, rw2_ref, rb2_ref, rw3_ref, rb3_ref,
               o_ref, acc):
    i = pl.program_id(0)
    x = p_ref[0] + p_ref[1]
    mx = jnp.dot(x, gk_ref[...], preferred_element_type=jnp.float32)
    mx = mx + ib_ref[...]
    mi = mi_ref[...]
    z = jax.nn.sigmoid(mx[:, :D] + mi[:, :D])
    r = jax.nn.sigmoid(mx[:, D:2 * D] + mi[:, D:2 * D])
    hh = jnp.tanh(mx[:, 2 * D:] + r * mi[:, 2 * D:])
    ls = z * f_ref[...] + (1.0 - z) * hh
    gid = gid_ref[0, 0, :]
    mask_t = (lax.broadcasted_iota(jnp.int32, (G, NB), 0)
              == gid[None, :]).astype(jnp.float32)
    contrib = jnp.dot(mask_t, ls, preferred_element_type=jnp.float32)

    @pl.when(i == 0)
    def _():
        acc[...] = contrib

    @pl.when(i > 0)
    def _():
        acc[...] = acc[...] + contrib

    @pl.when(i == pl.num_programs(0) - 1)
    def _():
        pooled = acc[...]
        h1 = _selu(
            jnp.dot(pooled, rw1_ref[...], preferred_element_type=jnp.float32)
            + rb1_ref[...])
        h2 = _selu(
            jnp.dot(h1, rw2_ref[...], preferred_element_type=jnp.float32)
            + rb2_ref[...])
        o_ref[...] = (jnp.dot(h2, rw3_ref[...],
                              preferred_element_type=jnp.float32)
                      + rb3_ref[...])


def _tc_post(parts, mi, features, gids3, gk, ib, rw1, rb1, rw2, rb2,
             rw3p, rb3p):
    R = rw1.shape[1]
    return pl.pallas_call(
        _post_body,
        grid=(GRID,),
        in_specs=[
            pl.BlockSpec((2, NB, D), lambda i: (0, i, 0)),
            pl.BlockSpec((NB, 3 * D), lambda i: (i, 0)),
            pl.BlockSpec((NB, D), lambda i: (i, 0)),
            pl.BlockSpec((1, 1, NB), lambda i: (i, 0, 0)),
            pl.BlockSpec((D, 3 * D), lambda i: (0, 0)),
            pl.BlockSpec((1, 3 * D), lambda i: (0, 0)),
            pl.BlockSpec((D, R), lambda i: (0, 0)),
            pl.BlockSpec((1, R), lambda i: (0, 0)),
            pl.BlockSpec((R, R), lambda i: (0, 0)),
            pl.BlockSpec((1, R), lambda i: (0, 0)),
            pl.BlockSpec((R, D), lambda i: (0, 0)),
            pl.BlockSpec((1, D), lambda i: (0, 0)),
        ],
        out_specs=pl.BlockSpec((G, D), lambda i: (0, 0)),
        out_shape=jax.ShapeDtypeStruct((G, D), jnp.float32),
        scratch_shapes=[pltpu.VMEM((G, D), jnp.float32)],
    )(parts, mi, features, gids3, gk, ib, rw1, rb1, rw2, rb2, rw3p, rb3p)


@jax.jit
def kernel(features, graph_ids, edges_topology, W1, b1, gru_kernel,
           gru_rec_kernel, gru_bias, rW1, rb1, rW2, rb2, rW3, rb3):
    w_cat = jnp.concatenate([W1[:D], W1[D:], gru_rec_kernel], axis=1)
    b_cat = jnp.concatenate(
        [b1, jnp.zeros((D,), jnp.float32), gru_bias[1]]).reshape(1, 5 * D)
    ab_tab, mi = _tc_pre(features, w_cat, b_cat)

    first = jnp.concatenate(
        [edges_topology[0], jnp.zeros((_EPAD,), jnp.int32)])
    second_g = jnp.concatenate(
        [edges_topology[1], jnp.zeros((_EPAD,), jnp.int32)])
    second = jnp.concatenate(
        [edges_topology[1], jnp.full((_EPAD,), N, jnp.int32)])
    idxc = jnp.concatenate(
        [2 * first.reshape(-1, _CH), 2 * second_g.reshape(-1, _CH) + 1],
        axis=1).reshape(-1)
    zeros_nd = jnp.zeros((_NP, D), jnp.float32)
    parts = _sc_edge(idxc, second, ab_tab.reshape(2 * N, D), zeros_nd)

    gids3 = graph_ids.reshape(GRID, 1, NB)
    ib = gru_bias[0].reshape(1, 3 * D)
    rw3p = jnp.pad(rW3, ((0, 0), (0, D - rW3.shape[1])))
    rb3p = jnp.broadcast_to(rb3.reshape(1, 1), (1, D))
    out = _tc_post(parts, mi, features, gids3, gru_kernel, ib,
                   rW1, rb1.reshape(1, -1), rW2, rb2.reshape(1, -1),
                   rw3p, rb3p)
    return out[:, :1]
